# baseline (speedup 1.0000x reference)
import functools
import math

import jax
import jax.numpy as jnp
from jax.experimental import pallas as pl
from jax.experimental.pallas import tpu as pltpu

QUANT = 0.9
TA = 512
TB = 256


def _fused_kernel(
    na, nb, k_th,
    x_ref, wo_ref, bo_ref, wo2_ref, bo2_ref, wp_ref, bp_ref,
    o_ref,
    y_sc, key_sc,
):
    i = pl.program_id(0)

    @pl.when(i < na)
    def _phase_a():
        y_blk = jnp.dot(
            x_ref[...], wo_ref[...], preferred_element_type=jnp.float32
        ) + bo_ref[...][None, :]
        y_sc[:, pl.ds(2 * i * TA, TA)] = y_blk
        y_blk2 = jnp.dot(
            x_ref[...], wo2_ref[...], preferred_element_type=jnp.float32
        ) + bo2_ref[...][None, :]
        y_sc[:, pl.ds((2 * i + 1) * TA, TA)] = y_blk2

    @pl.when(i >= na)
    def _phase_b():
        j = i - na
        z_blk = jnp.dot(
            y_sc[...], wp_ref[...], preferred_element_type=jnp.float32
        ) + bp_ref[...][None, :]
        u = jax.lax.bitcast_convert_type(z_blk, jnp.int32)
        key_sc[:, pl.ds(j * TB, TB)] = jnp.where(
            u >= 0, u, jnp.int32(-(2**31)) - u
        )

    @pl.when(i == na + nb - 1)
    def _epilogue():
        B = o_ref.shape[0]
        lo = jnp.full((B, 1), -(2**31), jnp.int32)
        hi = jnp.full((B, 1), 2**31 - 1, jnp.int32)

        def body(_, carry):
            lo, hi = carry
            mid = (lo >> 1) + (hi >> 1) + (lo & hi & 1)
            cnt = jnp.sum(
                (key_sc[...] <= mid).astype(jnp.int32), axis=1, keepdims=True
            )
            ge = cnt >= k_th
            lo = jnp.where(ge, lo, mid + 1)
            hi = jnp.where(ge, mid, hi)
            return lo, hi

        lo, hi = jax.lax.fori_loop(0, 32, body, (lo, hi))
        o_ref[...] = jnp.where(key_sc[...] > lo, y_sc[...], 0.0)


@jax.jit
def kernel(x, W_orig, b_orig, W_pol, b_pol):
    B, D = x.shape
    F = W_pol.shape[1]
    k_th = int(max(1, min(F, 1 + math.floor(QUANT * (F - 1)))))
    na = F // (2 * TA)
    nb = F // TB

    return pl.pallas_call(
        functools.partial(_fused_kernel, na, nb, k_th),
        grid=(na + nb,),
        in_specs=[
            pl.BlockSpec((B, D), lambda i: (0, 0)),
            pl.BlockSpec((D, TA), lambda i: (0, 2 * jnp.minimum(i, na - 1))),
            pl.BlockSpec((TA,), lambda i: (2 * jnp.minimum(i, na - 1),)),
            pl.BlockSpec((D, TA), lambda i: (0, 2 * jnp.minimum(i, na - 1) + 1)),
            pl.BlockSpec((TA,), lambda i: (2 * jnp.minimum(i, na - 1) + 1,)),
            pl.BlockSpec((F, TB), lambda i: (0, jnp.maximum(0, i - na))),
            pl.BlockSpec((TB,), lambda i: (jnp.maximum(0, i - na),)),
        ],
        out_specs=pl.BlockSpec((B, F), lambda i: (0, 0)),
        out_shape=jax.ShapeDtypeStruct((B, F), jnp.float32),
        scratch_shapes=[
            pltpu.VMEM((B, F), jnp.float32),
            pltpu.VMEM((B, F), jnp.int32),
        ],
        compiler_params=pltpu.CompilerParams(
            dimension_semantics=("arbitrary",),
            vmem_limit_bytes=100 * 1024 * 1024,
        ),
    )(x, W_orig, b_orig, W_orig, b_orig, W_pol, b_pol)


# speedup vs baseline: 1.0593x; 1.0593x over previous
import functools
import math

import jax
import jax.numpy as jnp
from jax.experimental import pallas as pl
from jax.experimental.pallas import tpu as pltpu

QUANT = 0.9
TA = 512
TB = 256


def _fused_kernel(
    na, nb, k_th,
    x_ref, wo_ref, bo_ref, wp_hbm, bp_ref,
    o_ref,
    y_sc, key_sc, wp_buf, sem,
):
    i = pl.program_id(0)

    def start_fetch(band, slot):
        pltpu.make_async_copy(
            wp_hbm.at[:, pl.ds(band * TB, TB)], wp_buf.at[slot], sem.at[slot]
        ).start()

    @pl.when(i < na)
    def _phase_a():
        y_blk = jnp.dot(
            x_ref[...], wo_ref[...], preferred_element_type=jnp.float32
        ) + bo_ref[...][None, :]
        y_sc[:, pl.ds(i * TA, TA)] = y_blk

    @pl.when(i == na - 2)
    def _warm0():
        start_fetch(0, 0)

    @pl.when(i == na - 1)
    def _warm1():
        start_fetch(1, 1)

    @pl.when(i >= na)
    def _phase_b():
        j = i - na
        slot = jax.lax.rem(j, 2)
        pltpu.make_async_copy(
            wp_hbm.at[:, pl.ds(j * TB, TB)], wp_buf.at[slot], sem.at[slot]
        ).wait()
        z_blk = jnp.dot(
            y_sc[...], wp_buf[slot], preferred_element_type=jnp.float32
        ) + bp_ref[...][None, :]
        u = jax.lax.bitcast_convert_type(z_blk, jnp.int32)
        key_sc[:, pl.ds(j * TB, TB)] = jnp.where(
            u >= 0, u, jnp.int32(-(2**31)) - u
        )

        @pl.when(j + 2 < nb)
        def _next():
            start_fetch(j + 2, slot)

    @pl.when(i == na + nb - 1)
    def _epilogue():
        B = o_ref.shape[0]
        lo = jnp.full((B, 1), -(2**31), jnp.int32)
        hi = jnp.full((B, 1), 2**31 - 1, jnp.int32)

        def body(_, carry):
            lo, hi = carry
            mid = (lo >> 1) + (hi >> 1) + (lo & hi & 1)
            cnt = jnp.sum(
                (key_sc[...] <= mid).astype(jnp.float32), axis=1, keepdims=True
            )
            ge = cnt >= jnp.float32(k_th)
            lo = jnp.where(ge, lo, mid + 1)
            hi = jnp.where(ge, mid, hi)
            return lo, hi

        lo, hi = jax.lax.fori_loop(0, 32, body, (lo, hi), unroll=16)
        o_ref[...] = jnp.where(key_sc[...] > lo, y_sc[...], 0.0)


@jax.jit
def kernel(x, W_orig, b_orig, W_pol, b_pol):
    B, D = x.shape
    F = W_pol.shape[1]
    k_th = int(max(1, min(F, 1 + math.floor(QUANT * (F - 1)))))
    na = F // TA
    nb = F // TB

    return pl.pallas_call(
        functools.partial(_fused_kernel, na, nb, k_th),
        grid=(na + nb,),
        in_specs=[
            pl.BlockSpec((B, D), lambda i: (0, 0)),
            pl.BlockSpec((D, TA), lambda i: (0, jnp.minimum(i, na - 1))),
            pl.BlockSpec((TA,), lambda i: (jnp.minimum(i, na - 1),)),
            pl.BlockSpec(memory_space=pl.ANY),
            pl.BlockSpec((TB,), lambda i: (jnp.maximum(0, i - na),)),
        ],
        out_specs=pl.BlockSpec((B, F), lambda i: (0, 0)),
        out_shape=jax.ShapeDtypeStruct((B, F), jnp.float32),
        scratch_shapes=[
            pltpu.VMEM((B, F), jnp.float32),
            pltpu.VMEM((B, F), jnp.int32),
            pltpu.VMEM((2, F, TB), jnp.float32),
            pltpu.SemaphoreType.DMA((2,)),
        ],
        compiler_params=pltpu.CompilerParams(
            dimension_semantics=("arbitrary",),
            vmem_limit_bytes=100 * 1024 * 1024,
        ),
    )(x, W_orig, b_orig, W_pol, b_pol)
